# baseline (speedup 1.0000x reference)
import functools
import math

import jax
import jax.numpy as jnp
from jax import lax
from jax.experimental import pallas as pl
from jax.experimental.pallas import tpu as pltpu
from jax.experimental.pallas import tpu_sc as plsc

D_MODEL = 64
SCALE = math.sqrt(D_MODEL)
N_IDX = 4096 * 200
NC, NS, LANES = 2, 16, 16
NW = NC * NS
BPW = N_IDX // NW
CHUNK = 128
NCHUNK = BPW // CHUNK
NBUF = 4
NGRP = NCHUNK // NBUF

_mesh = plsc.VectorSubcoreMesh(core_axis_name="c", subcore_axis_name="s")


@functools.partial(
    pl.kernel,
    mesh=_mesh,
    compiler_params=pltpu.CompilerParams(use_tc_tiling_on_sc=False),
    out_type=jax.ShapeDtypeStruct((N_IDX, D_MODEL), jnp.float32),
    scratch_types=[
        pltpu.VMEM((NCHUNK, CHUNK), jnp.int32),
        pltpu.VMEM((NBUF, CHUNK, D_MODEL), jnp.float32),
        pltpu.VMEM((NBUF, CHUNK, D_MODEL), jnp.float32),
        pltpu.SemaphoreType.DMA((NBUF,)),
        pltpu.SemaphoreType.DMA((NBUF,)),
    ],
)
def _emb_lookup(idx_hbm, lut_hbm, out_hbm, idx_v, gbuf, sbuf, gsem, ssem):
    wid = lax.axis_index("s") * NC + lax.axis_index("c")
    base = wid * BPW
    pltpu.sync_copy(idx_hbm.at[wid], idx_v)

    def start_gather(c, b):
        pltpu.make_async_copy(
            lut_hbm.at[idx_v.at[c]], gbuf.at[b], gsem.at[b]).start()

    def wait_gather(c, b):
        pltpu.make_async_copy(
            lut_hbm.at[idx_v.at[c]], gbuf.at[b], gsem.at[b]).wait()

    def out_slot(c):
        return out_hbm.at[pl.ds(pl.multiple_of(base + c * CHUNK, CHUNK), CHUNK)]

    def scale_chunk(b):
        def row_body(r, carry):
            for j in range(D_MODEL // LANES):
                sl = pl.ds(j * LANES, LANES)
                sbuf[b, r, sl] = gbuf[b, r, sl] * SCALE
            return carry
        lax.fori_loop(0, CHUNK, row_body, 0)

    for b in range(NBUF):
        start_gather(b, b)

    def group(g, carry):
        for b in range(NBUF):
            c = g * NBUF + b
            wait_gather(c, b)
            @pl.when(g > 0)
            def _():
                pltpu.make_async_copy(sbuf.at[b], out_slot(c - NBUF), ssem.at[b]).wait()
            scale_chunk(b)
            pltpu.make_async_copy(sbuf.at[b], out_slot(c), ssem.at[b]).start()
            @pl.when(g < NGRP - 1)
            def _():
                start_gather(c + NBUF, b)
        return carry

    lax.fori_loop(0, NGRP, group, 0)

    for b in range(NBUF):
        c = (NGRP - 1) * NBUF + b
        pltpu.make_async_copy(sbuf.at[b], out_slot(c), ssem.at[b]).wait()


def kernel(indices, lut):
    idx = indices.reshape(NW, NCHUNK, CHUNK).astype(jnp.int32)
    out = _emb_lookup(idx, lut)
    return out.reshape(indices.shape[0], indices.shape[1], D_MODEL)
